# Initial kernel scaffold; baseline (speedup 1.0000x reference)
import jax
import jax.numpy as jnp
from jax.experimental import pallas as pl


def kernel(x, wg, w1, b1, w2, b2, k):
    raise NotImplementedError("write your pallas kernel here")


# speedup vs baseline: 3.2809x; 3.2809x over previous
import functools

import jax
import jax.numpy as jnp
from jax import lax
from jax.experimental import pallas as pl
from jax.experimental.pallas import tpu as pltpu


def _gate_body(x_ref, wg_ref, gv_ref, gi_ref, ss_ref, ms_ref):
    i = pl.program_id(0)
    lg = lax.dot_general(x_ref[...].astype(jnp.bfloat16),
                         wg_ref[...].astype(jnp.bfloat16),
                         (((1,), (0,)), ((), ())),
                         preferred_element_type=jnp.float32)
    m = jnp.max(lg, axis=-1, keepdims=True)
    ex = jnp.exp(lg - m)
    p = ex / jnp.sum(ex, axis=-1, keepdims=True)
    e = p.shape[-1]
    i8 = lax.broadcasted_iota(jnp.int32, p.shape, 1)
    v1 = jnp.max(p, axis=-1, keepdims=True)
    a1 = jnp.min(jnp.where(p >= v1, i8, e), axis=-1)
    mask1 = i8 == a1[:, None]
    p2 = jnp.where(mask1, -1.0, p)
    v2 = jnp.max(p2, axis=-1, keepdims=True)
    a2 = jnp.min(jnp.where(p2 >= v2, i8, e), axis=-1)
    gv_ref[...] = jnp.concatenate([v1, v2], axis=-1)
    gi_ref[...] = jnp.concatenate([a1[:, None], a2[:, None]], axis=-1)

    @pl.when(i == 0)
    def _():
        ss_ref[...] = jnp.zeros_like(ss_ref)
        ms_ref[...] = jnp.zeros_like(ms_ref)

    ss_ref[...] += jnp.sum(p, axis=0, keepdims=True)
    ms_ref[...] += jnp.sum(mask1.astype(jnp.float32), axis=0, keepdims=True)


def _gate(x, wg):
    t, d = x.shape
    e = wg.shape[1]
    tb = min(512, t)
    grid = (t // tb,)
    return pl.pallas_call(
        _gate_body,
        grid=grid,
        in_specs=[
            pl.BlockSpec((tb, d), lambda i: (i, 0)),
            pl.BlockSpec((d, e), lambda i: (0, 0)),
        ],
        out_specs=[
            pl.BlockSpec((tb, 2), lambda i: (i, 0)),
            pl.BlockSpec((tb, 2), lambda i: (i, 0)),
            pl.BlockSpec((1, e), lambda i: (0, 0)),
            pl.BlockSpec((1, e), lambda i: (0, 0)),
        ],
        out_shape=[
            jax.ShapeDtypeStruct((t, 2), jnp.float32),
            jax.ShapeDtypeStruct((t, 2), jnp.int32),
            jax.ShapeDtypeStruct((1, e), jnp.float32),
            jax.ShapeDtypeStruct((1, e), jnp.float32),
        ],
        compiler_params=pltpu.CompilerParams(
            dimension_semantics=("arbitrary",)),
    )(x, wg)


def _ffn_body(nh, eob_ref, act_ref, xs_ref, w1_ref, b1_ref, w2_ref, b2_ref,
              g_ref, eo_ref):
    rb = pl.program_id(0)
    hb = pl.program_id(1)
    active = act_ref[rb] > 0

    @pl.when(active)
    def _():
        xb = xs_ref[...].astype(jnp.bfloat16)
        w1b = w1_ref[0].astype(jnp.bfloat16)
        h = lax.dot_general(xb, w1b, (((1,), (0,)), ((), ())),
                            preferred_element_type=jnp.float32)
        h = jnp.maximum(h + b1_ref[0], 0.0).astype(jnp.bfloat16)
        pt = lax.dot_general(h, w2_ref[0].astype(jnp.bfloat16),
                             (((1,), (0,)), ((), ())),
                             preferred_element_type=jnp.float32)

        @pl.when(hb == 0)
        def _():
            eo_ref[...] = pt

        @pl.when(hb > 0)
        def _():
            eo_ref[...] += pt

        @pl.when(hb == nh - 1)
        def _():
            g = g_ref[0, 0, :][:, None]
            eo_ref[...] = g * (eo_ref[...] + b2_ref[0])


def _ffn(eob, act, xs, w1, b1, w2, b2, g3, blk, nh):
    npad, d = xs.shape
    e, _, h = w1.shape
    hblk = h // nh
    nb = npad // blk
    grid_spec = pltpu.PrefetchScalarGridSpec(
        num_scalar_prefetch=2,
        grid=(nb, nh),
        in_specs=[
            pl.BlockSpec((blk, d), lambda rb, hb, eob, act: (rb, 0)),
            pl.BlockSpec((1, d, hblk),
                         lambda rb, hb, eob, act: (eob[rb], 0, hb * act[rb])),
            pl.BlockSpec((1, 1, hblk),
                         lambda rb, hb, eob, act: (eob[rb], 0, hb * act[rb])),
            pl.BlockSpec((1, hblk, d),
                         lambda rb, hb, eob, act: (eob[rb], hb * act[rb], 0)),
            pl.BlockSpec((1, 1, d),
                         lambda rb, hb, eob, act: (eob[rb], 0, 0)),
            pl.BlockSpec((1, 1, blk), lambda rb, hb, eob, act: (rb, 0, 0)),
        ],
        out_specs=pl.BlockSpec((blk, d), lambda rb, hb, eob, act: (rb, 0)),
    )
    return pl.pallas_call(
        functools.partial(_ffn_body, nh),
        grid_spec=grid_spec,
        out_shape=jax.ShapeDtypeStruct((npad, d), jnp.float32),
        compiler_params=pltpu.CompilerParams(
            dimension_semantics=("arbitrary", "arbitrary")),
    )(eob, act, xs, w1, b1, w2, b2, g3)


def kernel(x, wg, w1, b1, w2, b2, k):
    t, d = x.shape
    e = wg.shape[1]
    blk = 512 if t >= 4096 else 64
    nh = 8
    npad = 2 * t + e * blk
    nb = npad // blk

    gvals, gidx, ssum, msum = _gate(x, wg)

    flat_e = jnp.concatenate([gidx[:, 0], gidx[:, 1]])
    order = jnp.argsort(flat_e, stable=True)
    e_sorted = flat_e[order]
    counts = jnp.zeros((e,), jnp.int32).at[flat_e].add(1)
    cum = jnp.concatenate([jnp.zeros((1,), jnp.int32),
                           jnp.cumsum(counts)])[:e]
    pc = ((counts + blk - 1) // blk) * blk
    base = jnp.concatenate([jnp.zeros((1,), jnp.int32),
                            jnp.cumsum(pc)])[:e]
    i2 = jnp.arange(2 * t, dtype=jnp.int32)
    dest = base[e_sorted] + (i2 - cum[e_sorted])
    tok = (order % t).astype(jnp.int32)
    slot = (order // t).astype(jnp.int32)
    src = jnp.zeros((npad,), jnp.int32).at[dest].set(tok)
    gflat = jnp.concatenate([gvals[:, 0], gvals[:, 1]])
    grow = jnp.zeros((npad,), jnp.float32).at[dest].set(gflat[order])
    pos = jnp.zeros((t, 2), jnp.int32).at[tok, slot].set(dest)
    nbe = pc // blk
    eob = jnp.repeat(jnp.arange(e, dtype=jnp.int32), nbe,
                     total_repeat_length=nb)
    used = jnp.sum(nbe)
    act = (jnp.arange(nb, dtype=jnp.int32) < used).astype(jnp.int32)

    xs = jnp.take(x, src, axis=0)

    g3 = grow.reshape(nb, 1, blk)
    eo = _ffn(eob, act, xs, w1, b1, w2, b2, g3, blk, nh)

    out = jnp.take(eo, pos[:, 0], axis=0) + jnp.take(eo, pos[:, 1], axis=0)

    l_aux = e * jnp.sum((ssum[0] / t) * (msum[0] / t))
    return out, l_aux
